# baseline (speedup 1.0000x reference)
import functools

import jax
import jax.numpy as jnp
from jax import lax
from jax.experimental import pallas as pl
from jax.experimental.pallas import tpu as pltpu
from jax.experimental.pallas import tpu_sc as plsc

FACTOR = 32
PACK = 4
CHUNK = 128


def _gather_sc(batch, n_packed, width):
    info = plsc.get_sparse_core_info()
    nc, ns = info.num_cores, info.num_subcores
    nw = nc * ns
    b_per_w = batch // nw
    nchunk = b_per_w // CHUNK
    mesh = plsc.VectorSubcoreMesh(core_axis_name="c", subcore_axis_name="s")

    @functools.partial(
        pl.kernel,
        mesh=mesh,
        out_type=[
            jax.ShapeDtypeStruct((batch // PACK, PACK * FACTOR), jnp.float32),
            jax.ShapeDtypeStruct((batch // PACK, PACK * FACTOR), jnp.float32),
        ],
        scratch_types=[
            pltpu.VMEM((nchunk, CHUNK), jnp.int32),
            pltpu.VMEM((nchunk, CHUNK), jnp.int32),
            pltpu.VMEM((nchunk, CHUNK), jnp.int32),
            pltpu.VMEM((nchunk, CHUNK), jnp.int32),
            pltpu.VMEM((CHUNK, 128), jnp.float32),
            pltpu.VMEM((CHUNK, 128), jnp.float32),
            pltpu.VMEM((CHUNK // PACK, 128), jnp.float32),
            pltpu.VMEM((CHUNK // PACK, 128), jnp.float32),
            pltpu.SemaphoreType.DMA,
        ],
        compiler_params=pltpu.CompilerParams(needs_layout_passes=False),
    )
    def gather_kernel(eu_hbm, ei_hbm, uidx_hbm, iidx_hbm, uoff_hbm, ioff_hbm,
                      u_out, i_out,
                      uidx_v, iidx_v, uoff_v, ioff_v, ustage, istage,
                      urows_v, irows_v, sem):
        wid = lax.axis_index("s") * nc + lax.axis_index("c")
        base = wid * (b_per_w // PACK)
        pltpu.sync_copy(uidx_hbm.at[wid], uidx_v)
        pltpu.sync_copy(iidx_hbm.at[wid], iidx_v)
        pltpu.sync_copy(uoff_hbm.at[wid], uoff_v)
        pltpu.sync_copy(ioff_hbm.at[wid], ioff_v)
        cols_lo = lax.iota(jnp.int32, 16)
        cols_hi = cols_lo + 16
        rows_per_chunk = CHUNK // PACK
        for j in range(nchunk):
            cu = pltpu.async_copy(eu_hbm.at[uidx_v.at[j]], ustage, sem)
            ci = pltpu.async_copy(ei_hbm.at[iidx_v.at[j]], istage, sem)
            cu.wait()
            ci.wait()
            js = jnp.full((16,), j, jnp.int32)

            def row_body(r, _):
                rs = jnp.full((16,), r, jnp.int32)
                offu = plsc.load_gather(uoff_v, [js, rs])
                offi = plsc.load_gather(ioff_v, [js, rs])
                ro = r // PACK
                co = (r % PACK) * FACTOR
                urows_v[ro, pl.ds(co, 16)] = plsc.load_gather(
                    ustage, [rs, offu + cols_lo])
                urows_v[ro, pl.ds(co + 16, 16)] = plsc.load_gather(
                    ustage, [rs, offu + cols_hi])
                irows_v[ro, pl.ds(co, 16)] = plsc.load_gather(
                    istage, [rs, offi + cols_lo])
                irows_v[ro, pl.ds(co + 16, 16)] = plsc.load_gather(
                    istage, [rs, offi + cols_hi])
                return 0

            lax.fori_loop(0, CHUNK, row_body, 0)
            pltpu.sync_copy(urows_v,
                            u_out.at[pl.ds(base + j * rows_per_chunk,
                                           rows_per_chunk)])
            pltpu.sync_copy(irows_v,
                            i_out.at[pl.ds(base + j * rows_per_chunk,
                                           rows_per_chunk)])

    return gather_kernel, nw, nchunk


def _mlp_body(u_ref, i_ref, w0_ref, b0_ref, w1_ref, b1_ref, wo_ref, bo_ref,
              out_ref):
    u = u_ref[...]
    i = i_ref[...]
    w0 = w0_ref[...]
    x = jnp.dot(u, w0[:FACTOR, :], preferred_element_type=jnp.float32)
    x += jnp.dot(i, w0[FACTOR:, :], preferred_element_type=jnp.float32)
    x = jnp.maximum(x + b0_ref[...], 0.0)
    x = jnp.dot(x, w1_ref[...], preferred_element_type=jnp.float32)
    x = jnp.maximum(x + b1_ref[...], 0.0)
    pred = jnp.sum(x * wo_ref[...], axis=1) + bo_ref[0, 0]
    out_ref[...] = pred


@jax.jit
def kernel(user, item, embed_user, embed_item, W0, b0, W1, b1, Wo, bo):
    batch = user.shape[0]
    n_rows = embed_user.shape[0]
    gather_kernel, nw, nchunk = _gather_sc(batch, n_rows // PACK, 128)

    eu2 = embed_user.reshape(n_rows // PACK, PACK * FACTOR)
    ei2 = embed_item.reshape(n_rows // PACK, PACK * FACTOR)
    user = user.astype(jnp.int32)
    item = item.astype(jnp.int32)
    uidx = (user // PACK).reshape(nw, nchunk, CHUNK)
    iidx = (item // PACK).reshape(nw, nchunk, CHUNK)
    uoff = ((user % PACK) * FACTOR).reshape(nw, nchunk, CHUNK)
    ioff = ((item % PACK) * FACTOR).reshape(nw, nchunk, CHUNK)
    u_rows, i_rows = gather_kernel(eu2, ei2, uidx, iidx, uoff, ioff)
    u_rows = u_rows.reshape(batch, FACTOR)
    i_rows = i_rows.reshape(batch, FACTOR)

    blk = 2048
    grid = (batch // blk,)
    out = pl.pallas_call(
        _mlp_body,
        grid=grid,
        in_specs=[
            pl.BlockSpec((blk, FACTOR), lambda i: (i, 0)),
            pl.BlockSpec((blk, FACTOR), lambda i: (i, 0)),
            pl.BlockSpec(W0.shape, lambda i: (0, 0)),
            pl.BlockSpec((1, W0.shape[1]), lambda i: (0, 0)),
            pl.BlockSpec(W1.shape, lambda i: (0, 0)),
            pl.BlockSpec((1, W1.shape[1]), lambda i: (0, 0)),
            pl.BlockSpec((1, Wo.shape[0]), lambda i: (0, 0)),
            pl.BlockSpec((1, 1), lambda i: (0, 0)),
        ],
        out_specs=pl.BlockSpec((blk,), lambda i: (i,)),
        out_shape=jax.ShapeDtypeStruct((batch,), jnp.float32),
    )(u_rows, i_rows, W0, b0.reshape(1, -1), W1, b1.reshape(1, -1),
      Wo.reshape(1, -1), bo.reshape(1, 1))
    return out


# speedup vs baseline: 2.1492x; 2.1492x over previous
import functools

import jax
import jax.numpy as jnp
from jax import lax
from jax.experimental import pallas as pl
from jax.experimental.pallas import tpu as pltpu
from jax.experimental.pallas import tpu_sc as plsc

FACTOR = 32
BLK = 128
PACK = 4
G = 4
CHUNK = 128


def _gather_sc(batch, n_rows):
    info = plsc.get_sparse_core_info()
    nc, ns = info.num_cores, info.num_subcores
    nw = nc * ns
    b_per_w = batch // nw
    nchunk = b_per_w // CHUNK
    ngrp = b_per_w // G
    mesh = plsc.VectorSubcoreMesh(core_axis_name="c", subcore_axis_name="s")

    @functools.partial(
        pl.kernel,
        mesh=mesh,
        out_type=[
            jax.ShapeDtypeStruct((batch // PACK, PACK * FACTOR), jnp.float32),
            jax.ShapeDtypeStruct((batch // PACK, PACK * FACTOR), jnp.float32),
        ],
        scratch_types=[
            pltpu.VMEM((nchunk, CHUNK), jnp.int32),
            pltpu.VMEM((nchunk, CHUNK), jnp.int32),
            pltpu.VMEM((nchunk, CHUNK), jnp.int32),
            pltpu.VMEM((nchunk, CHUNK), jnp.int32),
            pltpu.VMEM((2, G, FACTOR, BLK), jnp.float32),
            pltpu.VMEM((b_per_w // PACK, 128), jnp.float32),
            pltpu.VMEM((b_per_w // PACK, 128), jnp.float32),
            pltpu.SemaphoreType.DMA,
            pltpu.SemaphoreType.DMA,
        ],
        compiler_params=pltpu.CompilerParams(needs_layout_passes=False),
    )
    def gather_kernel(euT_hbm, eiT_hbm, ubk_hbm, ibk_hbm, uco_hbm, ico_hbm,
                      u_out, i_out,
                      ubk_v, ibk_v, uco_v, ico_v, bufs,
                      urows_v, irows_v, sem0, sem1):
        wid = lax.axis_index("s") * nc + lax.axis_index("c")
        base = wid * (b_per_w // PACK)
        pltpu.sync_copy(ubk_hbm.at[wid], ubk_v)
        pltpu.sync_copy(ibk_hbm.at[wid], ibk_v)
        pltpu.sync_copy(uco_hbm.at[wid], uco_v)
        pltpu.sync_copy(ico_hbm.at[wid], ico_v)
        feat_lo = lax.iota(jnp.int32, 16)
        feat_hi = feat_lo + 16

        def phase(tab_hbm, bk_v, co_v, rows_v):
            def fire(g):
                par = jnp.bitwise_and(g, 1)

                def fire_k(k, _):
                    r = g * G + k
                    jj = jnp.full((16,), r // CHUNK, jnp.int32)
                    rr = jnp.full((16,), r % CHUNK, jnp.int32)
                    blk = jnp.max(plsc.load_gather(bk_v, [jj, rr]))
                    start = pl.multiple_of(blk * BLK, BLK)
                    cp0 = pltpu.make_async_copy(
                        tab_hbm.at[:, pl.ds(start, BLK)],
                        bufs.at[0, k], sem0)
                    cp1 = pltpu.make_async_copy(
                        tab_hbm.at[:, pl.ds(start, BLK)],
                        bufs.at[1, k], sem1)

                    @pl.when(par == 0)
                    def _():
                        cp0.start()

                    @pl.when(par == 1)
                    def _():
                        cp1.start()
                    return 0

                lax.fori_loop(0, G, fire_k, 0)

            def drain(g):
                par = jnp.bitwise_and(g, 1)

                def wait_k(k, _):
                    @pl.when(par == 0)
                    def _():
                        pltpu.make_async_copy(
                            tab_hbm.at[:, pl.ds(0, BLK)],
                            bufs.at[0, 0], sem0).wait()

                    @pl.when(par == 1)
                    def _():
                        pltpu.make_async_copy(
                            tab_hbm.at[:, pl.ds(0, BLK)],
                            bufs.at[1, 0], sem1).wait()
                    return 0

                lax.fori_loop(0, G, wait_k, 0)

            def extract(g):
                par = jnp.bitwise_and(g, 1)

                def ex_k(k, _):
                    r = g * G + k
                    jj = jnp.full((16,), r // CHUNK, jnp.int32)
                    rr = jnp.full((16,), r % CHUNK, jnp.int32)
                    co = plsc.load_gather(co_v, [jj, rr])
                    ps = jnp.full((16,), par, jnp.int32)
                    ks = jnp.full((16,), k, jnp.int32)
                    lo = plsc.load_gather(bufs, [ps, ks, feat_lo, co])
                    hi = plsc.load_gather(bufs, [ps, ks, feat_hi, co])
                    ro = r // PACK
                    cc = (r % PACK) * FACTOR
                    rows_v[ro, pl.ds(cc, 16)] = lo
                    rows_v[ro, pl.ds(cc + 16, 16)] = hi
                    return 0

                lax.fori_loop(0, G, ex_k, 0)

            def grp_body(g, _):
                fire(g)
                drain(g)
                extract(g)
                return 0

            lax.fori_loop(0, ngrp, grp_body, 0)

        phase(euT_hbm, ubk_v, uco_v, urows_v)
        phase(eiT_hbm, ibk_v, ico_v, irows_v)
        pltpu.sync_copy(urows_v, u_out.at[pl.ds(base, b_per_w // PACK)])
        pltpu.sync_copy(irows_v, i_out.at[pl.ds(base, b_per_w // PACK)])

    return gather_kernel, nw, nchunk


def _mlp_body(u_ref, i_ref, w0_ref, b0_ref, w1_ref, b1_ref, wo_ref, bo_ref,
              out_ref):
    u = u_ref[...]
    i = i_ref[...]
    w0 = w0_ref[...]
    x = jnp.dot(u, w0[:FACTOR, :], preferred_element_type=jnp.float32)
    x += jnp.dot(i, w0[FACTOR:, :], preferred_element_type=jnp.float32)
    x = jnp.maximum(x + b0_ref[...], 0.0)
    x = jnp.dot(x, w1_ref[...], preferred_element_type=jnp.float32)
    x = jnp.maximum(x + b1_ref[...], 0.0)
    pred = jnp.sum(x * wo_ref[...], axis=1) + bo_ref[0, 0]
    out_ref[...] = pred


@jax.jit
def kernel(user, item, embed_user, embed_item, W0, b0, W1, b1, Wo, bo):
    batch = user.shape[0]
    n_rows = embed_user.shape[0]
    gather_kernel, nw, nchunk = _gather_sc(batch, n_rows)

    euT = embed_user.T
    eiT = embed_item.T
    user = user.astype(jnp.int32)
    item = item.astype(jnp.int32)
    ubk = (user // BLK).reshape(nw, nchunk, CHUNK)
    ibk = (item // BLK).reshape(nw, nchunk, CHUNK)
    uco = (user % BLK).reshape(nw, nchunk, CHUNK)
    ico = (item % BLK).reshape(nw, nchunk, CHUNK)
    u_rows, i_rows = gather_kernel(euT, eiT, ubk, ibk, uco, ico)
    u_rows = u_rows.reshape(batch, FACTOR)
    i_rows = i_rows.reshape(batch, FACTOR)

    blk = 2048
    grid = (batch // blk,)
    out = pl.pallas_call(
        _mlp_body,
        grid=grid,
        in_specs=[
            pl.BlockSpec((blk, FACTOR), lambda i: (i, 0)),
            pl.BlockSpec((blk, FACTOR), lambda i: (i, 0)),
            pl.BlockSpec(W0.shape, lambda i: (0, 0)),
            pl.BlockSpec((1, W0.shape[1]), lambda i: (0, 0)),
            pl.BlockSpec(W1.shape, lambda i: (0, 0)),
            pl.BlockSpec((1, W1.shape[1]), lambda i: (0, 0)),
            pl.BlockSpec((1, Wo.shape[0]), lambda i: (0, 0)),
            pl.BlockSpec((1, 1), lambda i: (0, 0)),
        ],
        out_specs=pl.BlockSpec((blk,), lambda i: (i,)),
        out_shape=jax.ShapeDtypeStruct((batch,), jnp.float32),
    )(u_rows, i_rows, W0, b0.reshape(1, -1), W1, b1.reshape(1, -1),
      Wo.reshape(1, -1), bo.reshape(1, 1))
    return out


# speedup vs baseline: 2.6857x; 1.2496x over previous
import functools

import jax
import jax.numpy as jnp
from jax import lax
from jax.experimental import pallas as pl
from jax.experimental.pallas import tpu as pltpu
from jax.experimental.pallas import tpu_sc as plsc

FACTOR = 32
BLK = 128
PACK = 4
G = 8
CHUNK = 128


def _gather_sc(batch, n_rows):
    info = plsc.get_sparse_core_info()
    nc, ns = info.num_cores, info.num_subcores
    nw = nc * ns
    b_per_w = batch // nw
    nchunk = b_per_w // CHUNK
    ngrp = b_per_w // G
    mesh = plsc.VectorSubcoreMesh(core_axis_name="c", subcore_axis_name="s")

    @functools.partial(
        pl.kernel,
        mesh=mesh,
        out_type=[
            jax.ShapeDtypeStruct((batch // PACK, PACK * FACTOR), jnp.float32),
            jax.ShapeDtypeStruct((batch // PACK, PACK * FACTOR), jnp.float32),
        ],
        scratch_types=[
            pltpu.VMEM((nchunk, CHUNK), jnp.int32),
            pltpu.VMEM((nchunk, CHUNK), jnp.int32),
            pltpu.VMEM((nchunk, CHUNK), jnp.int32),
            pltpu.VMEM((nchunk, CHUNK), jnp.int32),
            pltpu.VMEM((2, G, FACTOR, BLK), jnp.float32),
            pltpu.VMEM((b_per_w // PACK, 128), jnp.float32),
            pltpu.VMEM((b_per_w // PACK, 128), jnp.float32),
            pltpu.SemaphoreType.DMA,
            pltpu.SemaphoreType.DMA,
        ],
        compiler_params=pltpu.CompilerParams(needs_layout_passes=False),
    )
    def gather_kernel(euT_hbm, eiT_hbm, ubk_hbm, ibk_hbm, uco_hbm, ico_hbm,
                      u_out, i_out,
                      ubk_v, ibk_v, uco_v, ico_v, bufs,
                      urows_v, irows_v, sem0, sem1):
        wid = lax.axis_index("s") * nc + lax.axis_index("c")
        base = wid * (b_per_w // PACK)
        pltpu.sync_copy(ubk_hbm.at[wid], ubk_v)
        pltpu.sync_copy(ibk_hbm.at[wid], ibk_v)
        pltpu.sync_copy(uco_hbm.at[wid], uco_v)
        pltpu.sync_copy(ico_hbm.at[wid], ico_v)
        feat_lo = lax.iota(jnp.int32, 16)
        feat_hi = feat_lo + 16

        def phase(tab_hbm, bk_v, co_v, rows_v):
            def fire(g):
                par = jnp.bitwise_and(g, 1)

                def fire_k(k, _):
                    r = g * G + k
                    jj = jnp.full((16,), r // CHUNK, jnp.int32)
                    rr = jnp.full((16,), r % CHUNK, jnp.int32)
                    blk = jnp.max(plsc.load_gather(bk_v, [jj, rr]))
                    start = pl.multiple_of(blk * BLK, BLK)
                    cp0 = pltpu.make_async_copy(
                        tab_hbm.at[:, pl.ds(start, BLK)],
                        bufs.at[0, k], sem0)
                    cp1 = pltpu.make_async_copy(
                        tab_hbm.at[:, pl.ds(start, BLK)],
                        bufs.at[1, k], sem1)

                    @pl.when(par == 0)
                    def _():
                        cp0.start()

                    @pl.when(par == 1)
                    def _():
                        cp1.start()
                    return 0

                lax.fori_loop(0, G, fire_k, 0)

            def drain(g):
                par = jnp.bitwise_and(g, 1)

                def wait_k(k, _):
                    @pl.when(par == 0)
                    def _():
                        pltpu.make_async_copy(
                            tab_hbm.at[:, pl.ds(0, BLK)],
                            bufs.at[0, k], sem0).wait()

                    @pl.when(par == 1)
                    def _():
                        pltpu.make_async_copy(
                            tab_hbm.at[:, pl.ds(0, BLK)],
                            bufs.at[1, k], sem1).wait()
                    return 0

                lax.fori_loop(0, G, wait_k, 0)

            def extract(g):
                par = jnp.bitwise_and(g, 1)

                def ex_k(k, _):
                    r = g * G + k
                    jj = jnp.full((16,), r // CHUNK, jnp.int32)
                    rr = jnp.full((16,), r % CHUNK, jnp.int32)
                    co = plsc.load_gather(co_v, [jj, rr])
                    ps = jnp.full((16,), par, jnp.int32)
                    ks = jnp.full((16,), k, jnp.int32)
                    lo = plsc.load_gather(bufs, [ps, ks, feat_lo, co])
                    hi = plsc.load_gather(bufs, [ps, ks, feat_hi, co])
                    ro = r // PACK
                    cc = (r % PACK) * FACTOR
                    rows_v[ro, pl.ds(cc, 16)] = lo
                    rows_v[ro, pl.ds(cc + 16, 16)] = hi
                    return 0

                lax.fori_loop(0, G, ex_k, 0)

            def grp_body(g, _):
                fire(g)
                drain(g)
                extract(g)
                return 0

            lax.fori_loop(0, ngrp, grp_body, 0)

        phase(euT_hbm, ubk_v, uco_v, urows_v)
        phase(eiT_hbm, ibk_v, ico_v, irows_v)
        pltpu.sync_copy(urows_v, u_out.at[pl.ds(base, b_per_w // PACK)])
        pltpu.sync_copy(irows_v, i_out.at[pl.ds(base, b_per_w // PACK)])

    return gather_kernel, nw, nchunk


def _mlp_body(u_ref, i_ref, w0_ref, b0_ref, w1_ref, b1_ref, wo_ref, bo_ref,
              out_ref):
    u = u_ref[...]
    i = i_ref[...]
    w0 = w0_ref[...]
    x = jnp.dot(u, w0[:FACTOR, :], preferred_element_type=jnp.float32)
    x += jnp.dot(i, w0[FACTOR:, :], preferred_element_type=jnp.float32)
    x = jnp.maximum(x + b0_ref[...], 0.0)
    x = jnp.dot(x, w1_ref[...], preferred_element_type=jnp.float32)
    x = jnp.maximum(x + b1_ref[...], 0.0)
    pred = jnp.sum(x * wo_ref[...], axis=1) + bo_ref[0, 0]
    out_ref[...] = pred


@jax.jit
def kernel(user, item, embed_user, embed_item, W0, b0, W1, b1, Wo, bo):
    batch = user.shape[0]
    n_rows = embed_user.shape[0]
    gather_kernel, nw, nchunk = _gather_sc(batch, n_rows)

    euT = embed_user.T
    eiT = embed_item.T
    user = user.astype(jnp.int32)
    item = item.astype(jnp.int32)
    ubk = (user // BLK).reshape(nw, nchunk, CHUNK)
    ibk = (item // BLK).reshape(nw, nchunk, CHUNK)
    uco = (user % BLK).reshape(nw, nchunk, CHUNK)
    ico = (item % BLK).reshape(nw, nchunk, CHUNK)
    u_rows, i_rows = gather_kernel(euT, eiT, ubk, ibk, uco, ico)
    u_rows = u_rows.reshape(batch, FACTOR)
    i_rows = i_rows.reshape(batch, FACTOR)

    blk = 2048
    grid = (batch // blk,)
    out = pl.pallas_call(
        _mlp_body,
        grid=grid,
        in_specs=[
            pl.BlockSpec((blk, FACTOR), lambda i: (i, 0)),
            pl.BlockSpec((blk, FACTOR), lambda i: (i, 0)),
            pl.BlockSpec(W0.shape, lambda i: (0, 0)),
            pl.BlockSpec((1, W0.shape[1]), lambda i: (0, 0)),
            pl.BlockSpec(W1.shape, lambda i: (0, 0)),
            pl.BlockSpec((1, W1.shape[1]), lambda i: (0, 0)),
            pl.BlockSpec((1, Wo.shape[0]), lambda i: (0, 0)),
            pl.BlockSpec((1, 1), lambda i: (0, 0)),
        ],
        out_specs=pl.BlockSpec((blk,), lambda i: (i,)),
        out_shape=jax.ShapeDtypeStruct((batch,), jnp.float32),
    )(u_rows, i_rows, W0, b0.reshape(1, -1), W1, b1.reshape(1, -1),
      Wo.reshape(1, -1), bo.reshape(1, 1))
    return out


# speedup vs baseline: 2.9290x; 1.0906x over previous
import functools

import jax
import jax.numpy as jnp
from jax import lax
from jax.experimental import pallas as pl
from jax.experimental.pallas import tpu as pltpu
from jax.experimental.pallas import tpu_sc as plsc

FACTOR = 32
BLK = 128
PACK = 4
G = 8
CHUNK = 128


def _gather_sc(batch, n_rows):
    info = plsc.get_sparse_core_info()
    nc, ns = info.num_cores, info.num_subcores
    nw = nc * ns
    b_per_w = batch // nw
    nchunk = b_per_w // CHUNK
    ngrp = b_per_w // G
    mesh = plsc.VectorSubcoreMesh(core_axis_name="c", subcore_axis_name="s")

    @functools.partial(
        pl.kernel,
        mesh=mesh,
        out_type=[
            jax.ShapeDtypeStruct((batch // PACK, PACK * FACTOR), jnp.float32),
            jax.ShapeDtypeStruct((batch // PACK, PACK * FACTOR), jnp.float32),
        ],
        scratch_types=[
            pltpu.VMEM((nchunk, CHUNK), jnp.int32),
            pltpu.VMEM((nchunk, CHUNK), jnp.int32),
            pltpu.VMEM((nchunk, CHUNK), jnp.int32),
            pltpu.VMEM((nchunk, CHUNK), jnp.int32),
            pltpu.VMEM((2, G, FACTOR, BLK), jnp.float32),
            pltpu.VMEM((b_per_w // PACK, 128), jnp.float32),
            pltpu.VMEM((b_per_w // PACK, 128), jnp.float32),
            pltpu.SemaphoreType.DMA,
            pltpu.SemaphoreType.DMA,
        ],
        compiler_params=pltpu.CompilerParams(needs_layout_passes=False),
    )
    def gather_kernel(euT_hbm, eiT_hbm, ubk_hbm, ibk_hbm, uco_hbm, ico_hbm,
                      u_out, i_out,
                      ubk_v, ibk_v, uco_v, ico_v, bufs,
                      urows_v, irows_v, sem0, sem1):
        wid = lax.axis_index("s") * nc + lax.axis_index("c")
        base = wid * (b_per_w // PACK)
        pltpu.sync_copy(ubk_hbm.at[wid], ubk_v)
        pltpu.sync_copy(ibk_hbm.at[wid], ibk_v)
        pltpu.sync_copy(uco_hbm.at[wid], uco_v)
        pltpu.sync_copy(ico_hbm.at[wid], ico_v)
        feat_lo = lax.iota(jnp.int32, 16)
        feat_hi = feat_lo + 16

        def phase(tab_hbm, bk_v, co_v, rows_v):
            def fire(g):
                par = jnp.bitwise_and(g, 1)

                def fire_k(k, _):
                    r = g * G + k
                    jj = jnp.full((16,), r // CHUNK, jnp.int32)
                    rr = jnp.full((16,), r % CHUNK, jnp.int32)
                    blk = jnp.max(plsc.load_gather(bk_v, [jj, rr]))
                    start = pl.multiple_of(blk * BLK, BLK)
                    cp0 = pltpu.make_async_copy(
                        tab_hbm.at[:, pl.ds(start, BLK)],
                        bufs.at[0, k], sem0)
                    cp1 = pltpu.make_async_copy(
                        tab_hbm.at[:, pl.ds(start, BLK)],
                        bufs.at[1, k], sem1)

                    @pl.when(par == 0)
                    def _():
                        cp0.start()

                    @pl.when(par == 1)
                    def _():
                        cp1.start()
                    return 0

                lax.fori_loop(0, G, fire_k, 0)

            def drain(g):
                par = jnp.bitwise_and(g, 1)

                def wait_k(k, _):
                    @pl.when(par == 0)
                    def _():
                        pltpu.make_async_copy(
                            tab_hbm.at[:, pl.ds(0, BLK)],
                            bufs.at[0, k], sem0).wait()

                    @pl.when(par == 1)
                    def _():
                        pltpu.make_async_copy(
                            tab_hbm.at[:, pl.ds(0, BLK)],
                            bufs.at[1, k], sem1).wait()
                    return 0

                lax.fori_loop(0, G, wait_k, 0)

            def extract(g):
                par = jnp.bitwise_and(g, 1)

                def ex_k(k, _):
                    r = g * G + k
                    jj = jnp.full((16,), r // CHUNK, jnp.int32)
                    rr = jnp.full((16,), r % CHUNK, jnp.int32)
                    co = plsc.load_gather(co_v, [jj, rr])
                    ps = jnp.full((16,), par, jnp.int32)
                    ks = jnp.full((16,), k, jnp.int32)
                    lo = plsc.load_gather(bufs, [ps, ks, feat_lo, co])
                    hi = plsc.load_gather(bufs, [ps, ks, feat_hi, co])
                    ro = r // PACK
                    cc = (r % PACK) * FACTOR
                    rows_v[ro, pl.ds(cc, 16)] = lo
                    rows_v[ro, pl.ds(cc + 16, 16)] = hi
                    return 0

                lax.fori_loop(0, G, ex_k, 0)

            fire(0)

            def grp_body(g, _):
                drain(g)

                @pl.when(g + 1 < ngrp)
                def _():
                    fire(g + 1)

                extract(g)
                return 0

            lax.fori_loop(0, ngrp, grp_body, 0)

        phase(euT_hbm, ubk_v, uco_v, urows_v)
        phase(eiT_hbm, ibk_v, ico_v, irows_v)
        pltpu.sync_copy(urows_v, u_out.at[pl.ds(base, b_per_w // PACK)])
        pltpu.sync_copy(irows_v, i_out.at[pl.ds(base, b_per_w // PACK)])

    return gather_kernel, nw, nchunk


def _mlp_body(u_ref, i_ref, w0_ref, b0_ref, w1_ref, b1_ref, wo_ref, bo_ref,
              out_ref):
    u = u_ref[...]
    i = i_ref[...]
    w0 = w0_ref[...]
    x = jnp.dot(u, w0[:FACTOR, :], preferred_element_type=jnp.float32)
    x += jnp.dot(i, w0[FACTOR:, :], preferred_element_type=jnp.float32)
    x = jnp.maximum(x + b0_ref[...], 0.0)
    x = jnp.dot(x, w1_ref[...], preferred_element_type=jnp.float32)
    x = jnp.maximum(x + b1_ref[...], 0.0)
    pred = jnp.sum(x * wo_ref[...], axis=1) + bo_ref[0, 0]
    out_ref[...] = pred


@jax.jit
def kernel(user, item, embed_user, embed_item, W0, b0, W1, b1, Wo, bo):
    batch = user.shape[0]
    n_rows = embed_user.shape[0]
    gather_kernel, nw, nchunk = _gather_sc(batch, n_rows)

    euT = embed_user.T
    eiT = embed_item.T
    user = user.astype(jnp.int32)
    item = item.astype(jnp.int32)
    ubk = (user // BLK).reshape(nw, nchunk, CHUNK)
    ibk = (item // BLK).reshape(nw, nchunk, CHUNK)
    uco = (user % BLK).reshape(nw, nchunk, CHUNK)
    ico = (item % BLK).reshape(nw, nchunk, CHUNK)
    u_rows, i_rows = gather_kernel(euT, eiT, ubk, ibk, uco, ico)
    u_rows = u_rows.reshape(batch, FACTOR)
    i_rows = i_rows.reshape(batch, FACTOR)

    blk = 2048
    grid = (batch // blk,)
    out = pl.pallas_call(
        _mlp_body,
        grid=grid,
        in_specs=[
            pl.BlockSpec((blk, FACTOR), lambda i: (i, 0)),
            pl.BlockSpec((blk, FACTOR), lambda i: (i, 0)),
            pl.BlockSpec(W0.shape, lambda i: (0, 0)),
            pl.BlockSpec((1, W0.shape[1]), lambda i: (0, 0)),
            pl.BlockSpec(W1.shape, lambda i: (0, 0)),
            pl.BlockSpec((1, W1.shape[1]), lambda i: (0, 0)),
            pl.BlockSpec((1, Wo.shape[0]), lambda i: (0, 0)),
            pl.BlockSpec((1, 1), lambda i: (0, 0)),
        ],
        out_specs=pl.BlockSpec((blk,), lambda i: (i,)),
        out_shape=jax.ShapeDtypeStruct((batch,), jnp.float32),
    )(u_rows, i_rows, W0, b0.reshape(1, -1), W1, b1.reshape(1, -1),
      Wo.reshape(1, -1), bo.reshape(1, 1))
    return out


# speedup vs baseline: 3.1925x; 1.0900x over previous
import functools

import jax
import jax.numpy as jnp
from jax import lax
from jax.experimental import pallas as pl
from jax.experimental.pallas import tpu as pltpu
from jax.experimental.pallas import tpu_sc as plsc

FACTOR = 32
BLK = 128
PACK = 4
G = 8
CHUNK = 128


def _gather_sc(batch, n_rows):
    info = plsc.get_sparse_core_info()
    nc, ns = info.num_cores, info.num_subcores
    nw = nc * ns
    b_per_w = batch // nw
    nchunk = b_per_w // CHUNK
    ngrp = b_per_w // G
    mesh = plsc.VectorSubcoreMesh(core_axis_name="c", subcore_axis_name="s")

    @functools.partial(
        pl.kernel,
        mesh=mesh,
        out_type=[
            jax.ShapeDtypeStruct((batch // PACK, PACK * FACTOR), jnp.float32),
            jax.ShapeDtypeStruct((batch // PACK, PACK * FACTOR), jnp.float32),
        ],
        scratch_types=[
            pltpu.VMEM((nchunk, CHUNK), jnp.int32),
            pltpu.VMEM((nchunk, CHUNK), jnp.int32),
            pltpu.VMEM((nchunk, CHUNK), jnp.int32),
            pltpu.VMEM((nchunk, CHUNK), jnp.int32),
            pltpu.VMEM((2, G, FACTOR, BLK), jnp.float32),
            pltpu.VMEM((b_per_w // PACK, 128), jnp.float32),
            pltpu.VMEM((b_per_w // PACK, 128), jnp.float32),
            pltpu.SemaphoreType.DMA,
            pltpu.SemaphoreType.DMA,
        ],
        compiler_params=pltpu.CompilerParams(needs_layout_passes=False),
    )
    def gather_kernel(euT_hbm, eiT_hbm, ubk_hbm, ibk_hbm, uco_hbm, ico_hbm,
                      u_out, i_out,
                      ubk_v, ibk_v, uco_v, ico_v, bufs,
                      urows_v, irows_v, sem0, sem1):
        wid = lax.axis_index("s") * nc + lax.axis_index("c")
        base = wid * (b_per_w // PACK)
        pltpu.sync_copy(ubk_hbm.at[wid], ubk_v)
        pltpu.sync_copy(ibk_hbm.at[wid], ibk_v)
        pltpu.sync_copy(uco_hbm.at[wid], uco_v)
        pltpu.sync_copy(ico_hbm.at[wid], ico_v)
        feat_lo = lax.iota(jnp.int32, 16)
        feat_hi = feat_lo + 16

        def phase(tab_hbm, bk_v, co_v, rows_v):
            def fire(g):
                par = jnp.bitwise_and(g, 1)

                def fire_k(k, _):
                    r = g * G + k
                    jj = jnp.full((16,), r // CHUNK, jnp.int32)
                    rr = jnp.full((16,), r % CHUNK, jnp.int32)
                    blk = jnp.max(plsc.load_gather(bk_v, [jj, rr]))
                    start = pl.multiple_of(blk * BLK, BLK)
                    cp0 = pltpu.make_async_copy(
                        tab_hbm.at[:, pl.ds(start, BLK)],
                        bufs.at[0, k], sem0)
                    cp1 = pltpu.make_async_copy(
                        tab_hbm.at[:, pl.ds(start, BLK)],
                        bufs.at[1, k], sem1)

                    @pl.when(par == 0)
                    def _():
                        cp0.start()

                    @pl.when(par == 1)
                    def _():
                        cp1.start()
                    return 0

                lax.fori_loop(0, G, fire_k, 0)

            def drain(g):
                par = jnp.bitwise_and(g, 1)

                def wait_k(k, _):
                    @pl.when(par == 0)
                    def _():
                        pltpu.make_async_copy(
                            tab_hbm.at[:, pl.ds(0, BLK)],
                            bufs.at[0, k], sem0).wait()

                    @pl.when(par == 1)
                    def _():
                        pltpu.make_async_copy(
                            tab_hbm.at[:, pl.ds(0, BLK)],
                            bufs.at[1, k], sem1).wait()
                    return 0

                lax.fori_loop(0, G, wait_k, 0)

            def extract(g):
                par = jnp.bitwise_and(g, 1)

                def ex_k(k, _):
                    r = g * G + k
                    jj = jnp.full((16,), r // CHUNK, jnp.int32)
                    rr = jnp.full((16,), r % CHUNK, jnp.int32)
                    co = plsc.load_gather(co_v, [jj, rr])
                    ps = jnp.full((16,), par, jnp.int32)
                    ks = jnp.full((16,), k, jnp.int32)
                    lo = plsc.load_gather(bufs, [ps, ks, feat_lo, co])
                    hi = plsc.load_gather(bufs, [ps, ks, feat_hi, co])
                    ro = r // PACK
                    cc = (r % PACK) * FACTOR
                    rows_v[ro, pl.ds(cc, 16)] = lo
                    rows_v[ro, pl.ds(cc + 16, 16)] = hi
                    return 0

                lax.fori_loop(0, G, ex_k, 0)

            fire(0)

            def grp_body(g, _):
                drain(g)

                @pl.when(g + 1 < ngrp)
                def _():
                    fire(g + 1)

                extract(g)
                return 0

            lax.fori_loop(0, ngrp, grp_body, 0)

        phase(euT_hbm, ubk_v, uco_v, urows_v)
        phase(eiT_hbm, ibk_v, ico_v, irows_v)
        pltpu.sync_copy(urows_v, u_out.at[pl.ds(base, b_per_w // PACK)])
        pltpu.sync_copy(irows_v, i_out.at[pl.ds(base, b_per_w // PACK)])

    return gather_kernel, nw, nchunk


def _mlp_body(u_ref, i_ref, w0u_ref, w0i_ref, b0_ref, w1_ref, b1_ref,
              wo_ref, bo_ref, out_ref):
    x = jnp.dot(u_ref[...], w0u_ref[...], preferred_element_type=jnp.float32)
    x += jnp.dot(i_ref[...], w0i_ref[...], preferred_element_type=jnp.float32)
    x = jnp.maximum(x + b0_ref[...], 0.0)
    x = jnp.dot(x, w1_ref[...], preferred_element_type=jnp.float32)
    x = jnp.maximum(x + b1_ref[...], 0.0)
    out_ref[...] = (
        jnp.dot(x, wo_ref[...], preferred_element_type=jnp.float32)
        + bo_ref[0, 0])


@jax.jit
def kernel(user, item, embed_user, embed_item, W0, b0, W1, b1, Wo, bo):
    batch = user.shape[0]
    n_rows = embed_user.shape[0]
    gather_kernel, nw, nchunk = _gather_sc(batch, n_rows)

    euT = embed_user.T
    eiT = embed_item.T
    user = user.astype(jnp.int32)
    item = item.astype(jnp.int32)
    ubk = (user // BLK).reshape(nw, nchunk, CHUNK)
    ibk = (item // BLK).reshape(nw, nchunk, CHUNK)
    uco = (user % BLK).reshape(nw, nchunk, CHUNK)
    ico = (item % BLK).reshape(nw, nchunk, CHUNK)
    u_rows, i_rows = gather_kernel(euT, eiT, ubk, ibk, uco, ico)

    eye = jnp.eye(PACK, dtype=jnp.float32)
    w0u = jnp.kron(eye, W0[:FACTOR, :])
    w0i = jnp.kron(eye, W0[FACTOR:, :])
    w1 = jnp.kron(eye, W1)
    wo = jnp.kron(eye, Wo)
    b0r = jnp.tile(b0, PACK).reshape(1, -1)
    b1r = jnp.tile(b1, PACK).reshape(1, -1)

    bp = batch // PACK
    blk = bp // 2
    grid = (bp // blk,)
    out = pl.pallas_call(
        _mlp_body,
        grid=grid,
        in_specs=[
            pl.BlockSpec((blk, 128), lambda i: (i, 0)),
            pl.BlockSpec((blk, 128), lambda i: (i, 0)),
            pl.BlockSpec(w0u.shape, lambda i: (0, 0)),
            pl.BlockSpec(w0i.shape, lambda i: (0, 0)),
            pl.BlockSpec((1, b0r.shape[1]), lambda i: (0, 0)),
            pl.BlockSpec(w1.shape, lambda i: (0, 0)),
            pl.BlockSpec((1, b1r.shape[1]), lambda i: (0, 0)),
            pl.BlockSpec(wo.shape, lambda i: (0, 0)),
            pl.BlockSpec((1, 1), lambda i: (0, 0)),
        ],
        out_specs=pl.BlockSpec((blk, PACK), lambda i: (i, 0)),
        out_shape=jax.ShapeDtypeStruct((bp, PACK), jnp.float32),
    )(u_rows, i_rows, w0u, w0i, b0r, w1, b1r, wo, bo.reshape(1, 1))
    return out.reshape(batch)
